# Initial kernel scaffold; baseline (speedup 1.0000x reference)
import jax
import jax.numpy as jnp
from jax.experimental import pallas as pl


def kernel(pos, edge_index, offsets):
    raise NotImplementedError("write your pallas kernel here")


# speedup vs baseline: 150.3808x; 150.3808x over previous
import jax
import jax.numpy as jnp
from jax import lax
from jax.experimental import pallas as pl
from jax.experimental.pallas import tpu as pltpu
from jax.experimental.pallas import tpu_sc as plsc

NC, NS, L = 2, 16, 16
NW = NC * NS
BLK = 128
CB = 8
CHUNK = CB * BLK
PD = 8
SB = 8


def _make_body(e, nb):
    nblk = e // BLK
    maxb = (nblk + NW - 1) // NW
    niter = (maxb + CB - 1) // CB
    stage_iter = ((nb + NS - 1) // NS + SB - 1) // SB

    def body(pos_hbm, ei_hbm, off_hbm, out_hbm,
             sidx0, sidx1, didx0, didx1, offv0, offv1,
             srow0, srow1, drow0, drow1, outv0, outv1,
             tblk, rbuf, sh,
             gsem0, gsem1, outsem0, outsem1,
             isem0, isem1, osem0, osem1):
        sidx = (sidx0, sidx1)
        didx = (didx0, didx1)
        offv = (offv0, offv1)
        srow = (srow0, srow1)
        drow = (drow0, drow1)
        outv = (outv0, outv1)
        gsem = (gsem0, gsem1)
        outsem = (outsem0, outsem1)
        isem = (isem0, isem1)
        osem = (osem0, osem1)

        sid = lax.axis_index("s")
        wid = sid * NC + lax.axis_index("c")
        iota16 = lax.iota(jnp.int32, L)
        col = [jnp.full((L,), c, jnp.int32) for c in range(3)]

        slo = sid * nb // NS
        shi = (sid + 1) * nb // NS

        @pl.loop(0, stage_iter)
        def stage_loop(t):
            j0 = jnp.minimum(slo + t * SB, shi - SB)
            pltpu.sync_copy(pos_hbm.at[pl.ds(j0, SB)], tblk)
            for bi in range(SB):
                for g in range(BLK // L):
                    rows = iota16 + (bi * BLK + g * L)
                    for c in range(3):
                        x = tblk[bi, c, pl.ds(g * L, L)]
                        plsc.store_scatter(rbuf, [rows, col[c]], x)
            pltpu.sync_copy(rbuf, sh.at[pl.ds(j0 * BLK, SB * BLK), :])

        plsc.subcore_barrier()

        blo = wid * nblk // NW
        bhi = (wid + 1) * nblk // NW

        def chunk_start(k):
            return jnp.minimum(blo + k * CB, bhi - CB) * BLK

        def fire_idx(k, b):
            start = chunk_start(k)
            pltpu.async_copy(ei_hbm.at[pl.ds(start, CHUNK)], sidx[b], isem[b])
            pltpu.async_copy(ei_hbm.at[pl.ds(e + start, CHUNK)], didx[b], isem[b])

        def wait_idx(b):
            pltpu.make_async_copy(
                ei_hbm.at[pl.ds(0, CHUNK)], sidx[b], isem[b]).wait()
            pltpu.make_async_copy(
                ei_hbm.at[pl.ds(0, CHUNK)], didx[b], isem[b]).wait()

        def fire_off(k, b):
            pltpu.async_copy(
                off_hbm.at[pl.ds(chunk_start(k) * 4, CHUNK * 4)], offv[b], osem[b])

        def wait_off(b):
            pltpu.make_async_copy(
                off_hbm.at[pl.ds(0, CHUNK * 4)], offv[b], osem[b]).wait()

        def fire_gathers(b):
            pltpu.async_copy(sh.at[sidx[b]], srow[b], gsem[b])
            pltpu.async_copy(sh.at[didx[b]], drow[b], gsem[b])

        def wait_gathers(b):
            pltpu.make_async_copy(sh.at[sidx[b]], srow[b], gsem[b]).wait()
            pltpu.make_async_copy(sh.at[didx[b]], drow[b], gsem[b]).wait()

        def wait_out(b):
            pltpu.make_async_copy(
                outv[b], out_hbm.at[pl.ds(0, CHUNK * 4)], outsem[b]).wait()

        def compute(k, b):
            @plsc.parallel_loop(0, CB)
            def block_loop(j):
                rbase = j * BLK
                obase = j * (4 * BLK)
                for sub in range(BLK // L):
                    row = iota16 + (rbase + sub * L)
                    d = []
                    for c in range(3):
                        sv = plsc.load_gather(srow[b], [row, col[c]])
                        tv = plsc.load_gather(drow[b], [row, col[c]])
                        ov = offv[b][pl.ds(obase + c * BLK + sub * L, L)]
                        d.append(sv - tv + ov)
                    s = d[0] * d[0] + d[1] * d[1] + d[2] * d[2]
                    ii = plsc.bitcast(s, jnp.int32)
                    ii = jnp.int32(0x5F3759DF) - lax.shift_right_arithmetic(ii, 1)
                    y = plsc.bitcast(ii, jnp.float32)
                    h = s * jnp.float32(0.5)
                    y = y * (jnp.float32(1.5) - h * y * y)
                    y = y * (jnp.float32(1.5) - h * y * y)
                    dist = s * y
                    o = obase + sub * L
                    outv[b][pl.ds(o, L)] = dist
                    outv[b][pl.ds(o + BLK, L)] = d[0]
                    outv[b][pl.ds(o + 2 * BLK, L)] = d[1]
                    outv[b][pl.ds(o + 3 * BLK, L)] = d[2]

            pltpu.async_copy(
                outv[b], out_hbm.at[pl.ds(chunk_start(k) * 4, CHUNK * 4)],
                outsem[b])

        fire_idx(0, 0)
        fire_off(0, 0)
        fire_idx(1, 1)
        fire_off(1, 1)
        wait_idx(0)
        fire_gathers(0)

        @pl.loop(0, (niter + 1) // 2)
        def chunk_loop(t):
            for b in (0, 1):
                kk = t * 2 + b

                @pl.when(kk < niter)
                def _():
                    wait_gathers(b)

                    @pl.when(kk + 2 < niter)
                    def _():
                        fire_idx(kk + 2, b)

                    @pl.when(kk + 1 < niter)
                    def _():
                        wait_idx(1 - b)
                        fire_gathers(1 - b)

                    @pl.when(kk >= 2)
                    def _():
                        wait_out(b)

                    wait_off(b)
                    compute(kk, b)

                    @pl.when(kk + 2 < niter)
                    def _():
                        fire_off(kk + 2, b)

        wait_out((niter - 2) % 2)
        wait_out((niter - 1) % 2)

    return body


def kernel(pos, edge_index, offsets):
    e = edge_index.shape[1]
    n = pos.shape[0]
    nb = (n + BLK - 1) // BLK
    mesh = plsc.VectorSubcoreMesh(core_axis_name="c", subcore_axis_name="s")
    f = pl.kernel(
        _make_body(e, nb),
        out_type=jax.ShapeDtypeStruct((e * 4,), jnp.float32),
        mesh=mesh,
        compiler_params=pltpu.CompilerParams(
            needs_layout_passes=False, use_tc_tiling_on_sc=False),
        scratch_types=(
            [pltpu.VMEM((CHUNK,), jnp.int32)] * 4
            + [pltpu.VMEM((CHUNK * 4,), jnp.float32)] * 2
            + [pltpu.VMEM((CHUNK, PD), jnp.float32)] * 4
            + [pltpu.VMEM((CHUNK * 4,), jnp.float32)] * 2
            + [
                pltpu.VMEM((SB, 4, BLK), jnp.float32),
                pltpu.VMEM((SB * BLK, PD), jnp.float32),
                pltpu.VMEM_SHARED((nb * BLK, PD), jnp.float32),
            ]
            + [pltpu.SemaphoreType.DMA] * 8
        ),
    )
    pos_blocks = (
        jnp.pad(pos, ((0, nb * BLK - n), (0, 1)))
        .T.reshape(4, nb, BLK)
        .transpose(1, 0, 2)
    )
    off_blocks = (
        jnp.pad(offsets, ((0, 0), (0, 1)))
        .T.reshape(4, e // BLK, BLK)
        .transpose(1, 0, 2)
        .reshape(e * 4)
    )
    out_flat = f(pos_blocks, edge_index.reshape(2 * e), off_blocks)
    return out_flat.reshape(e // BLK, 4, BLK).transpose(0, 2, 1).reshape(e, 4)
